# baseline (speedup 1.0000x reference)
import functools

import jax
import jax.numpy as jnp
from jax import lax
from jax.experimental import pallas as pl
from jax.experimental.pallas import tpu as pltpu
from jax.experimental.pallas import tpu_sc as plsc

F32 = jnp.float32
I32 = jnp.int32

ATOM_DIMS_K = [119, 4, 12, 12, 10, 6, 6, 2, 2]
HID = 128
NGRAPH = 512
NNODES = 10000
NEDGES = 320000
BLK = 512
NB = 20
NP = NB * BLK
TDIM = 176
PAD_ID = 173

NW = 32
NS = 16
CH = 64
NCHUNK = 160
ND = 4
EPT = NCHUNK * CH
NE_PAD = NW * EPT
RPT = NP // NS


def _build_scatter():
    mesh = plsc.VectorSubcoreMesh(core_axis_name="c", subcore_axis_name="s",
                                  num_cores=2, num_subcores=NS)

    @functools.partial(
        pl.kernel,
        mesh=mesh,
        out_type=jax.ShapeDtypeStruct((2, NP, HID), F32),
        scratch_types=[
            pltpu.VMEM((ND, CH), I32),
            pltpu.VMEM((ND, CH), I32),
            pltpu.VMEM((ND, CH, HID), F32),
            pltpu.VMEM_SHARED((NP, HID), F32),
            pltpu.SemaphoreType.DMA((ND,)),
            pltpu.SemaphoreType.DMA((ND,)),
            pltpu.SemaphoreType.DMA((ND,)),
            pltpu.SemaphoreType.DMA((ND,)),
        ],
    )
    def scat(h_hbm, srcs_hbm, dsts_hbm, zeros_hbm, out_hbm,
             src_v, dst_v, rows_v, agg_sh, gsem, ssem, isem, dsem):
        c = lax.axis_index("c")
        s = lax.axis_index("s")
        wid = s * 2 + c
        pltpu.sync_copy(zeros_hbm.at[pl.ds(s * RPT, RPT)],
                        agg_sh.at[pl.ds(s * RPT, RPT)])
        plsc.subcore_barrier()

        def fire_idx(j, b):
            pltpu.async_copy(srcs_hbm.at[wid, pl.ds(j * CH, CH)],
                             src_v.at[b], isem.at[b])

        def wait_idx(j, b):
            pltpu.make_async_copy(srcs_hbm.at[wid, pl.ds(j * CH, CH)],
                                  src_v.at[b], isem.at[b]).wait()

        def fire_dst(j, b):
            pltpu.async_copy(dsts_hbm.at[wid, pl.ds(j * CH, CH)],
                             dst_v.at[b], dsem.at[b])

        def wait_dst(j, b):
            pltpu.make_async_copy(dsts_hbm.at[wid, pl.ds(j * CH, CH)],
                                  dst_v.at[b], dsem.at[b]).wait()

        def fire_gather(b):
            pltpu.async_copy(h_hbm.at[src_v.at[b]], rows_v.at[b], gsem.at[b])

        def wait_gather(b):
            pltpu.make_async_copy(h_hbm.at[src_v.at[b]], rows_v.at[b],
                                  gsem.at[b]).wait()

        def fire_scatter(b):
            pltpu.async_copy(rows_v.at[b], agg_sh.at[dst_v.at[b]], ssem.at[b],
                             add=True)

        def wait_scatter(b):
            pltpu.make_async_copy(rows_v.at[b], agg_sh.at[dst_v.at[b]],
                                  ssem.at[b]).wait()

        def chunk_step(j, b, drain_prev, tail):
            wait_gather(b)
            if not tail:
                fire_idx(j + ND, b)
            wait_dst(j, b)
            fire_scatter(b)
            if not tail or not isinstance(j, int) or j + 3 < NCHUNK:
                bp = (b + 3) % ND
                wait_idx(j + 3, bp)
                if drain_prev:
                    wait_scatter(bp)
                fire_dst(j + 3, bp)
                fire_gather(bp)

        pltpu.sync_copy(srcs_hbm.at[wid, pl.ds(0, CH)], src_v.at[0])
        fire_gather(0)
        fire_idx(1, 1)
        fire_idx(2, 2)
        fire_idx(3, 3)
        fire_dst(0, 0)
        fire_dst(1, 1)
        fire_dst(2, 2)
        wait_idx(1, 1)
        fire_gather(1)
        wait_idx(2, 2)
        fire_gather(2)
        chunk_step(0, 0, drain_prev=False, tail=False)
        chunk_step(1, 1, drain_prev=True, tail=False)
        chunk_step(2, 2, drain_prev=True, tail=False)
        chunk_step(3, 3, drain_prev=True, tail=False)

        def body(jq, _):
            j0 = jq * ND
            for b in range(ND):
                chunk_step(j0 + b, b, drain_prev=True, tail=False)
            return 0

        lax.fori_loop(1, NCHUNK // ND - 1, body, 0)

        t0 = NCHUNK - ND
        chunk_step(t0 + 0, 0, drain_prev=True, tail=True)
        chunk_step(t0 + 1, 1, drain_prev=True, tail=True)
        chunk_step(t0 + 2, 2, drain_prev=True, tail=True)
        chunk_step(t0 + 3, 3, drain_prev=True, tail=True)
        for b in range(ND):
            wait_scatter(b)
        plsc.subcore_barrier()
        pltpu.sync_copy(agg_sh.at[pl.ds(s * RPT, RPT)],
                        out_hbm.at[c, pl.ds(s * RPT, RPT)])

    return scat


_SCATTER_CACHE = []


def _scatter_add(*args):
    if not _SCATTER_CACHE:
        _SCATTER_CACHE.append(_build_scatter())
    return _SCATTER_CACHE[0](*args)


def _hdot(a, b, dims=None):
    dn = (((1,), (0,)), ((), ())) if dims is None else dims
    return lax.dot_general(a, b, dimension_numbers=dn,
                           preferred_element_type=F32,
                           precision=lax.Precision.HIGHEST)


def _ddot(a, b):
    return jnp.dot(a, b, preferred_element_type=F32)


def _bn(x, m_ref, s_ref, b_ref):
    return (x - m_ref[...]) * s_ref[...] + b_ref[...]


def _embed_call(ids_pad, table):
    def body(ids_ref, tab_ref, out_ref):
        ids = ids_ref[...]
        iota = lax.broadcasted_iota(I32, (BLK, TDIM), 1)
        a = jnp.zeros((BLK, TDIM), F32)
        for col in range(9):
            a += (ids[:, col][:, None] == iota).astype(F32)
        out_ref[...] = _hdot(a, tab_ref[...])

    return pl.pallas_call(
        body,
        grid=(NB,),
        in_specs=[pl.BlockSpec((BLK, 16), lambda i: (i, 0)),
                  pl.BlockSpec((TDIM, HID), lambda i: (0, 0))],
        out_specs=pl.BlockSpec((BLK, HID), lambda i: (i, 0)),
        out_shape=jax.ShapeDtypeStruct((NP, HID), F32),
    )(ids_pad, table)


def _onehot(b):
    iota = lax.broadcasted_iota(I32, (BLK, NGRAPH), 1)
    return (b[:, None] == iota).astype(F32)


_FULL2 = lambda i: (0, 0)
_BN_SPECS = [pl.BlockSpec((1, HID), _FULL2)] * 3


def _mlp_call(h, agg, batch3, eps1, w1, b1, bn1, w2, b2, bn2, relu_out,
              cls_w=None):
    with_cls = cls_w is not None

    def body(h_ref, agg_ref, b_ref, eps_ref, w1_ref, b1_ref, m1, s1, t1,
             w2_ref, b2_ref, m2, s2, t2, *rest):
        if with_cls:
            cw1_ref, cb1_ref, cw2_ref, cb2_ref = rest[:4]
            hout_ref, pool_ref, cls_ref = rest[4:]
        else:
            hout_ref, pool_ref = rest
        i = pl.program_id(0)
        u = h_ref[...] * eps_ref[...] + agg_ref[0] + agg_ref[1]
        t = _bn(_ddot(u, w1_ref[...]) + b1_ref[...], m1, s1, t1)
        t = jnp.maximum(t, 0.0)
        w = _bn(_ddot(t, w2_ref[...]) + b2_ref[...], m2, s2, t2)
        if relu_out:
            w = jnp.maximum(w, 0.0)
        hout_ref[...] = w
        a = _onehot(b_ref[0, 0])
        contrib = _hdot(a, w, dims=(((0,), (0,)), ((), ())))

        @pl.when(i == 0)
        def _():
            pool_ref[...] = contrib

        @pl.when(i != 0)
        def _():
            pool_ref[...] += contrib

        if with_cls:
            @pl.when(i == NB - 1)
            def _():
                g = pool_ref[...]
                tt = jnp.maximum(_ddot(g, cw1_ref[...]) + cb1_ref[...], 0.0)
                cls_ref[...] = _ddot(tt, cw2_ref[...]) + cb2_ref[...]

    in_specs = [
        pl.BlockSpec((BLK, HID), lambda i: (i, 0)),
        pl.BlockSpec((2, BLK, HID), lambda i: (0, i, 0)),
        pl.BlockSpec((1, 1, BLK), lambda i: (i, 0, 0)),
        pl.BlockSpec((1, 1), _FULL2),
        pl.BlockSpec((HID, HID), _FULL2),
        pl.BlockSpec((1, HID), _FULL2),
        *_BN_SPECS,
        pl.BlockSpec((HID, HID), _FULL2),
        pl.BlockSpec((1, HID), _FULL2),
        *_BN_SPECS,
    ]
    out_specs = [
        pl.BlockSpec((BLK, HID), lambda i: (i, 0)),
        pl.BlockSpec((NGRAPH, HID), _FULL2),
    ]
    out_shape = [
        jax.ShapeDtypeStruct((NP, HID), F32),
        jax.ShapeDtypeStruct((NGRAPH, HID), F32),
    ]
    args = [h, agg, batch3, eps1, w1, b1, *bn1, w2, b2, *bn2]
    if with_cls:
        cw1, cb1, cw2, cb2 = cls_w
        in_specs += [pl.BlockSpec((HID, HID), _FULL2),
                     pl.BlockSpec((1, HID), _FULL2),
                     pl.BlockSpec((HID, 1), _FULL2),
                     pl.BlockSpec((1, 1), _FULL2)]
        out_specs.append(pl.BlockSpec((NGRAPH, 1), _FULL2))
        out_shape.append(jax.ShapeDtypeStruct((NGRAPH, 1), F32))
        args += [cw1, cb1, cw2, cb2]
    return pl.pallas_call(
        body,
        grid=(NB,),
        in_specs=in_specs,
        out_specs=out_specs,
        out_shape=out_shape,
    )(*args)


def _vn_bcast_call(pooled, vn_h, w1, b1, bn1, w2, b2, h_new, batch3):
    def body(pool_ref, vnh_ref, w1_ref, b1_ref, m1, s1, t1, w2_ref, b2_ref,
             hnew_ref, b_ref, hupd_ref, vnout_ref):
        i = pl.program_id(0)
        z = pool_ref[...] + vnh_ref[...]
        t = _bn(_ddot(z, w1_ref[...]) + b1_ref[...], m1, s1, t1)
        t = jnp.maximum(t, 0.0)
        vn_new = _ddot(t, w2_ref[...]) + b2_ref[...] + vnh_ref[...]
        a = _onehot(b_ref[0, 0])
        hupd_ref[...] = hnew_ref[...] + _hdot(a, vn_new)

        @pl.when(i == 0)
        def _():
            vnout_ref[...] = vn_new

    return pl.pallas_call(
        body,
        grid=(NB,),
        in_specs=[
            pl.BlockSpec((NGRAPH, HID), _FULL2),
            pl.BlockSpec((NGRAPH, HID), _FULL2),
            pl.BlockSpec((HID, HID), _FULL2),
            pl.BlockSpec((1, HID), _FULL2),
            *_BN_SPECS,
            pl.BlockSpec((HID, HID), _FULL2),
            pl.BlockSpec((1, HID), _FULL2),
            pl.BlockSpec((BLK, HID), lambda i: (i, 0)),
            pl.BlockSpec((1, 1, BLK), lambda i: (i, 0, 0)),
        ],
        out_specs=[
            pl.BlockSpec((BLK, HID), lambda i: (i, 0)),
            pl.BlockSpec((NGRAPH, HID), _FULL2),
        ],
        out_shape=[
            jax.ShapeDtypeStruct((NP, HID), F32),
            jax.ShapeDtypeStruct((NGRAPH, HID), F32),
        ],
    )(pooled, vn_h, w1, b1, *bn1, w2, b2, h_new, batch3)


def _bn_rows(bn):
    scale = bn["gamma"] * lax.rsqrt(bn["var"] + 1e-5)
    return (bn["mean"][None, :], scale[None, :], bn["beta"][None, :])


def kernel(x, edge_index, edge_attr, batch, params):
    del edge_attr
    offs = []
    acc = 0
    for d in ATOM_DIMS_K:
        offs.append(acc)
        acc += d
    ids = x.astype(I32) + jnp.asarray(offs, I32)[None, :]
    ids_pad = jnp.full((NP, 16), PAD_ID, I32).at[:NNODES, :9].set(ids)
    table = jnp.zeros((TDIM, HID), F32).at[:acc].set(
        jnp.concatenate(params["atom_emb"], axis=0))

    src = edge_index[0].astype(I32)
    dst = edge_index[1].astype(I32)
    srcs = jnp.zeros((NE_PAD,), I32).at[:NEDGES].set(src).reshape(NW, EPT)
    pad_dst = NNODES + jnp.arange(NE_PAD, dtype=I32) % (NP - NNODES)
    dsts = pad_dst.at[:NEDGES].set(dst).reshape(NW, EPT)

    batch3 = jnp.full((NP,), NGRAPH, I32).at[:NNODES].set(
        batch.astype(I32)).reshape(NB, 1, BLK)
    zeros = jnp.zeros((NP, HID), F32)

    h = _embed_call(ids_pad, table)
    vn_h = jnp.broadcast_to(params["vn_embedding"], (NGRAPH, HID))

    cls = None
    for l in range(3):
        cp = params["convs"][l]
        bn1 = _bn_rows(cp["bn"])
        bn2 = _bn_rows(params["bns"][l])
        eps1 = (1.0 + cp["eps"]).reshape(1, 1).astype(F32)

        agg = _scatter_add(h, srcs, dsts, zeros)

        if l < 2:
            h_new, pooled = _mlp_call(h, agg, batch3, eps1, cp["W1"],
                                      cp["b1"][None, :], bn1, cp["W2"],
                                      cp["b2"][None, :], bn2, relu_out=True)
            vp = params["vn_mlps"][l]
            h, vn_h = _vn_bcast_call(pooled, vn_h, vp["W1"],
                                     vp["b1"][None, :], _bn_rows(vp["bn"]),
                                     vp["W2"], vp["b2"][None, :], h_new,
                                     batch3)
        else:
            cl = params["classifier"]
            cls_w = (cl["W1"], cl["b1"][None, :], cl["W2"],
                     cl["b2"][None, :])
            _, _, cls = _mlp_call(h, agg, batch3, eps1, cp["W1"],
                                  cp["b1"][None, :], bn1, cp["W2"],
                                  cp["b2"][None, :], bn2, relu_out=False,
                                  cls_w=cls_w)
    return cls[:, 0]


# speedup vs baseline: 1.0657x; 1.0657x over previous
import functools

import jax
import jax.numpy as jnp
from jax import lax
from jax.experimental import pallas as pl
from jax.experimental.pallas import tpu as pltpu
from jax.experimental.pallas import tpu_sc as plsc

F32 = jnp.float32
I32 = jnp.int32

ATOM_DIMS_K = [119, 4, 12, 12, 10, 6, 6, 2, 2]
HID = 128
NGRAPH = 512
NNODES = 10000
NEDGES = 320000
BLK = 512
NB = 20
NP = NB * BLK
TDIM = 176
PAD_ID = 173

NW = 32
NS = 16
CH = 64
ND = 4
NCH0 = 260
NCH1 = 60
EPT_MAX = NCH0 * CH
NE_PAD = NS * (NCH0 + NCH1) * CH
RPT = NP // NS


def _build_scatter():
    mesh = plsc.VectorSubcoreMesh(core_axis_name="c", subcore_axis_name="s",
                                  num_cores=2, num_subcores=NS)

    @functools.partial(
        pl.kernel,
        mesh=mesh,
        out_type=jax.ShapeDtypeStruct((2, NP, HID), F32),
        scratch_types=[
            pltpu.VMEM((ND, CH), I32),
            pltpu.VMEM((ND, CH), I32),
            pltpu.VMEM((ND, CH, HID), F32),
            pltpu.VMEM_SHARED((NP, HID), F32),
            pltpu.SemaphoreType.DMA((ND,)),
            pltpu.SemaphoreType.DMA((ND,)),
            pltpu.SemaphoreType.DMA((ND,)),
            pltpu.SemaphoreType.DMA((ND,)),
        ],
    )
    def scat(h_hbm, srcs_hbm, dsts_hbm, zeros_hbm, out_hbm,
             src_v, dst_v, rows_v, agg_sh, gsem, ssem, isem, dsem):
        c = lax.axis_index("c")
        s = lax.axis_index("s")
        nch = jnp.where(c == 0, NCH0, NCH1)
        pltpu.sync_copy(zeros_hbm.at[pl.ds(s * RPT, RPT)],
                        agg_sh.at[pl.ds(s * RPT, RPT)])
        plsc.subcore_barrier()

        def fire_idx(j, b):
            pltpu.async_copy(srcs_hbm.at[c, s, pl.ds(j * CH, CH)],
                             src_v.at[b], isem.at[b])

        def wait_idx(j, b):
            pltpu.make_async_copy(srcs_hbm.at[c, s, pl.ds(j * CH, CH)],
                                  src_v.at[b], isem.at[b]).wait()

        def fire_dst(j, b):
            pltpu.async_copy(dsts_hbm.at[c, s, pl.ds(j * CH, CH)],
                             dst_v.at[b], dsem.at[b])

        def wait_dst(j, b):
            pltpu.make_async_copy(dsts_hbm.at[c, s, pl.ds(j * CH, CH)],
                                  dst_v.at[b], dsem.at[b]).wait()

        def fire_gather(b):
            pltpu.async_copy(h_hbm.at[src_v.at[b]], rows_v.at[b], gsem.at[b])

        def wait_gather(b):
            pltpu.make_async_copy(h_hbm.at[src_v.at[b]], rows_v.at[b],
                                  gsem.at[b]).wait()

        def fire_scatter(b):
            pltpu.async_copy(rows_v.at[b], agg_sh.at[dst_v.at[b]], ssem.at[b],
                             add=True)

        def wait_scatter(b):
            pltpu.make_async_copy(rows_v.at[b], agg_sh.at[dst_v.at[b]],
                                  ssem.at[b]).wait()

        def chunk_step(j, b, drain_prev, prefetch, launch):
            wait_gather(b)
            if prefetch:
                fire_idx(j + ND, b)
            wait_dst(j, b)
            fire_scatter(b)
            if launch:
                bp = (b + 3) % ND
                wait_idx(j + 3, bp)
                if drain_prev:
                    wait_scatter(bp)
                fire_dst(j + 3, bp)
                fire_gather(bp)

        pltpu.sync_copy(srcs_hbm.at[c, s, pl.ds(0, CH)], src_v.at[0])
        fire_gather(0)
        fire_idx(1, 1)
        fire_idx(2, 2)
        fire_idx(3, 3)
        fire_dst(0, 0)
        fire_dst(1, 1)
        fire_dst(2, 2)
        wait_idx(1, 1)
        fire_gather(1)
        wait_idx(2, 2)
        fire_gather(2)
        chunk_step(0, 0, drain_prev=False, prefetch=True, launch=True)
        chunk_step(1, 1, drain_prev=True, prefetch=True, launch=True)
        chunk_step(2, 2, drain_prev=True, prefetch=True, launch=True)
        chunk_step(3, 3, drain_prev=True, prefetch=True, launch=True)

        def body(jq, _):
            j0 = jq * ND
            for b in range(ND):
                chunk_step(j0 + b, b, drain_prev=True, prefetch=True,
                           launch=True)
            return 0

        lax.fori_loop(1, nch // ND - 1, body, 0)

        t0 = nch - ND
        chunk_step(t0 + 0, 0, drain_prev=True, prefetch=False, launch=True)
        chunk_step(t0 + 1, 1, drain_prev=True, prefetch=False, launch=False)
        chunk_step(t0 + 2, 2, drain_prev=True, prefetch=False, launch=False)
        chunk_step(t0 + 3, 3, drain_prev=True, prefetch=False, launch=False)
        for b in range(ND):
            wait_scatter(b)
        plsc.subcore_barrier()
        pltpu.sync_copy(agg_sh.at[pl.ds(s * RPT, RPT)],
                        out_hbm.at[c, pl.ds(s * RPT, RPT)])

    return scat


_SCATTER_CACHE = []


def _scatter_add(*args):
    if not _SCATTER_CACHE:
        _SCATTER_CACHE.append(_build_scatter())
    return _SCATTER_CACHE[0](*args)


def _hdot(a, b, dims=None):
    dn = (((1,), (0,)), ((), ())) if dims is None else dims
    return lax.dot_general(a, b, dimension_numbers=dn,
                           preferred_element_type=F32,
                           precision=lax.Precision.HIGHEST)


def _ddot(a, b):
    return jnp.dot(a, b, preferred_element_type=F32)


def _bn(x, m_ref, s_ref, b_ref):
    return (x - m_ref[...]) * s_ref[...] + b_ref[...]


def _embed_call(ids_pad, table):
    def body(ids_ref, tab_ref, out_ref):
        ids = ids_ref[...]
        iota = lax.broadcasted_iota(I32, (BLK, TDIM), 1)
        a = jnp.zeros((BLK, TDIM), F32)
        for col in range(9):
            a += (ids[:, col][:, None] == iota).astype(F32)
        out_ref[...] = _hdot(a, tab_ref[...])

    return pl.pallas_call(
        body,
        grid=(NB,),
        in_specs=[pl.BlockSpec((BLK, 16), lambda i: (i, 0)),
                  pl.BlockSpec((TDIM, HID), lambda i: (0, 0))],
        out_specs=pl.BlockSpec((BLK, HID), lambda i: (i, 0)),
        out_shape=jax.ShapeDtypeStruct((NP, HID), F32),
    )(ids_pad, table)


def _onehot(b):
    iota = lax.broadcasted_iota(I32, (BLK, NGRAPH), 1)
    return (b[:, None] == iota).astype(F32)


_FULL2 = lambda i: (0, 0)
_BN_SPECS = [pl.BlockSpec((1, HID), _FULL2)] * 3


def _mlp_call(h, agg, batch3, eps1, w1, b1, bn1, w2, b2, bn2, relu_out,
              cls_w=None):
    with_cls = cls_w is not None

    def body(h_ref, agg_ref, b_ref, eps_ref, w1_ref, b1_ref, m1, s1, t1,
             w2_ref, b2_ref, m2, s2, t2, *rest):
        if with_cls:
            cw1_ref, cb1_ref, cw2_ref, cb2_ref = rest[:4]
            hout_ref, pool_ref, cls_ref = rest[4:]
        else:
            hout_ref, pool_ref = rest
        i = pl.program_id(0)
        u = h_ref[...] * eps_ref[...] + agg_ref[0] + agg_ref[1]
        t = _bn(_ddot(u, w1_ref[...]) + b1_ref[...], m1, s1, t1)
        t = jnp.maximum(t, 0.0)
        w = _bn(_ddot(t, w2_ref[...]) + b2_ref[...], m2, s2, t2)
        if relu_out:
            w = jnp.maximum(w, 0.0)
        hout_ref[...] = w
        a = _onehot(b_ref[0, 0])
        contrib = _hdot(a, w, dims=(((0,), (0,)), ((), ())))

        @pl.when(i == 0)
        def _():
            pool_ref[...] = contrib

        @pl.when(i != 0)
        def _():
            pool_ref[...] += contrib

        if with_cls:
            @pl.when(i == NB - 1)
            def _():
                g = pool_ref[...]
                tt = jnp.maximum(_ddot(g, cw1_ref[...]) + cb1_ref[...], 0.0)
                cls_ref[...] = _ddot(tt, cw2_ref[...]) + cb2_ref[...]

    in_specs = [
        pl.BlockSpec((BLK, HID), lambda i: (i, 0)),
        pl.BlockSpec((2, BLK, HID), lambda i: (0, i, 0)),
        pl.BlockSpec((1, 1, BLK), lambda i: (i, 0, 0)),
        pl.BlockSpec((1, 1), _FULL2),
        pl.BlockSpec((HID, HID), _FULL2),
        pl.BlockSpec((1, HID), _FULL2),
        *_BN_SPECS,
        pl.BlockSpec((HID, HID), _FULL2),
        pl.BlockSpec((1, HID), _FULL2),
        *_BN_SPECS,
    ]
    out_specs = [
        pl.BlockSpec((BLK, HID), lambda i: (i, 0)),
        pl.BlockSpec((NGRAPH, HID), _FULL2),
    ]
    out_shape = [
        jax.ShapeDtypeStruct((NP, HID), F32),
        jax.ShapeDtypeStruct((NGRAPH, HID), F32),
    ]
    args = [h, agg, batch3, eps1, w1, b1, *bn1, w2, b2, *bn2]
    if with_cls:
        cw1, cb1, cw2, cb2 = cls_w
        in_specs += [pl.BlockSpec((HID, HID), _FULL2),
                     pl.BlockSpec((1, HID), _FULL2),
                     pl.BlockSpec((HID, 1), _FULL2),
                     pl.BlockSpec((1, 1), _FULL2)]
        out_specs.append(pl.BlockSpec((NGRAPH, 1), _FULL2))
        out_shape.append(jax.ShapeDtypeStruct((NGRAPH, 1), F32))
        args += [cw1, cb1, cw2, cb2]
    return pl.pallas_call(
        body,
        grid=(NB,),
        in_specs=in_specs,
        out_specs=out_specs,
        out_shape=out_shape,
    )(*args)


def _vn_bcast_call(pooled, vn_h, w1, b1, bn1, w2, b2, h_new, batch3):
    def body(pool_ref, vnh_ref, w1_ref, b1_ref, m1, s1, t1, w2_ref, b2_ref,
             hnew_ref, b_ref, hupd_ref, vnout_ref):
        i = pl.program_id(0)
        z = pool_ref[...] + vnh_ref[...]
        t = _bn(_ddot(z, w1_ref[...]) + b1_ref[...], m1, s1, t1)
        t = jnp.maximum(t, 0.0)
        vn_new = _ddot(t, w2_ref[...]) + b2_ref[...] + vnh_ref[...]
        a = _onehot(b_ref[0, 0])
        hupd_ref[...] = hnew_ref[...] + _hdot(a, vn_new)

        @pl.when(i == 0)
        def _():
            vnout_ref[...] = vn_new

    return pl.pallas_call(
        body,
        grid=(NB,),
        in_specs=[
            pl.BlockSpec((NGRAPH, HID), _FULL2),
            pl.BlockSpec((NGRAPH, HID), _FULL2),
            pl.BlockSpec((HID, HID), _FULL2),
            pl.BlockSpec((1, HID), _FULL2),
            *_BN_SPECS,
            pl.BlockSpec((HID, HID), _FULL2),
            pl.BlockSpec((1, HID), _FULL2),
            pl.BlockSpec((BLK, HID), lambda i: (i, 0)),
            pl.BlockSpec((1, 1, BLK), lambda i: (i, 0, 0)),
        ],
        out_specs=[
            pl.BlockSpec((BLK, HID), lambda i: (i, 0)),
            pl.BlockSpec((NGRAPH, HID), _FULL2),
        ],
        out_shape=[
            jax.ShapeDtypeStruct((NP, HID), F32),
            jax.ShapeDtypeStruct((NGRAPH, HID), F32),
        ],
    )(pooled, vn_h, w1, b1, *bn1, w2, b2, h_new, batch3)


def _bn_rows(bn):
    scale = bn["gamma"] * lax.rsqrt(bn["var"] + 1e-5)
    return (bn["mean"][None, :], scale[None, :], bn["beta"][None, :])


def kernel(x, edge_index, edge_attr, batch, params):
    del edge_attr
    offs = []
    acc = 0
    for d in ATOM_DIMS_K:
        offs.append(acc)
        acc += d
    ids = x.astype(I32) + jnp.asarray(offs, I32)[None, :]
    ids_pad = jnp.full((NP, 16), PAD_ID, I32).at[:NNODES, :9].set(ids)
    table = jnp.zeros((TDIM, HID), F32).at[:acc].set(
        jnp.concatenate(params["atom_emb"], axis=0))

    src = edge_index[0].astype(I32)
    dst = edge_index[1].astype(I32)
    src_pad = jnp.zeros((NE_PAD,), I32).at[:NEDGES].set(src)
    pad_dst = NNODES + jnp.arange(NE_PAD, dtype=I32) % (NP - NNODES)
    dst_pad = pad_dst.at[:NEDGES].set(dst)
    n0 = NS * NCH0 * CH
    srcs = jnp.zeros((2, NS, EPT_MAX), I32)
    srcs = srcs.at[0].set(src_pad[:n0].reshape(NS, EPT_MAX))
    srcs = srcs.at[1, :, :NCH1 * CH].set(
        src_pad[n0:].reshape(NS, NCH1 * CH))
    junk = NNODES + jnp.arange(2 * NS * EPT_MAX, dtype=I32) % (NP - NNODES)
    dsts = junk.reshape(2, NS, EPT_MAX)
    dsts = dsts.at[0].set(dst_pad[:n0].reshape(NS, EPT_MAX))
    dsts = dsts.at[1, :, :NCH1 * CH].set(
        dst_pad[n0:].reshape(NS, NCH1 * CH))

    batch3 = jnp.full((NP,), NGRAPH, I32).at[:NNODES].set(
        batch.astype(I32)).reshape(NB, 1, BLK)
    zeros = jnp.zeros((NP, HID), F32)

    h = _embed_call(ids_pad, table)
    vn_h = jnp.broadcast_to(params["vn_embedding"], (NGRAPH, HID))

    cls = None
    for l in range(3):
        cp = params["convs"][l]
        bn1 = _bn_rows(cp["bn"])
        bn2 = _bn_rows(params["bns"][l])
        eps1 = (1.0 + cp["eps"]).reshape(1, 1).astype(F32)

        agg = _scatter_add(h, srcs, dsts, zeros)

        if l < 2:
            h_new, pooled = _mlp_call(h, agg, batch3, eps1, cp["W1"],
                                      cp["b1"][None, :], bn1, cp["W2"],
                                      cp["b2"][None, :], bn2, relu_out=True)
            vp = params["vn_mlps"][l]
            h, vn_h = _vn_bcast_call(pooled, vn_h, vp["W1"],
                                     vp["b1"][None, :], _bn_rows(vp["bn"]),
                                     vp["W2"], vp["b2"][None, :], h_new,
                                     batch3)
        else:
            cl = params["classifier"]
            cls_w = (cl["W1"], cl["b1"][None, :], cl["W2"],
                     cl["b2"][None, :])
            _, _, cls = _mlp_call(h, agg, batch3, eps1, cp["W1"],
                                  cp["b1"][None, :], bn1, cp["W2"],
                                  cp["b2"][None, :], bn2, relu_out=False,
                                  cls_w=cls_w)
    return cls[:, 0]
